# baseline (speedup 1.0000x reference)
import functools

import jax
import jax.numpy as jnp
from jax import lax
from jax.experimental import pallas as pl
from jax.experimental.pallas import tpu as pltpu
from jax.experimental.pallas import tpu_sc as plsc

NC = 2
NS = 16
L = 16
NW = NC * NS
C = 16


def _emb_kernel(B, T, D, P, idx_hbm, wpe_hbm, wte_hbm, out_hbm,
                idx_v, wpe_v, rows_v, gsems, ssems, wsems):
    wid = lax.axis_index("s") * NC + lax.axis_index("c")
    pos_base = wid * P
    n_pc = P // C
    vecs_per_row = D // L

    for pc in range(n_pc):
        for b in range(B):
            row_base = b * T + pos_base + pc * C
            pltpu.sync_copy(idx_hbm.at[pl.ds(row_base, C)], idx_v.at[pc, b])

    wpe_copies = [None] * n_pc
    g_copies = [[None] * B for _ in range(n_pc)]
    s_copies = [[None] * B for _ in range(n_pc)]

    def start_chunk(pc):
        h = pc % 2
        wpe_copies[pc] = pltpu.async_copy(
            wpe_hbm.at[pl.ds(pos_base + pc * C, C)], wpe_v.at[h], wsems.at[h])
        for b in range(B):
            g_copies[pc][b] = pltpu.async_copy(
                wte_hbm.at[idx_v.at[pc, b]], rows_v.at[h, b], gsems.at[h, b])

    def add_chunk(h):
        def add_row(r, _):
            for v in range(vecs_per_row):
                sl = pl.ds(v * L, L)
                w = wpe_v[h, r, sl]
                for b in range(B):
                    rows_v[h, b, r, sl] = rows_v[h, b, r, sl] + w
            return _
        lax.fori_loop(0, C, add_row, 0)

    start_chunk(0)
    for pc in range(n_pc):
        h = pc % 2
        wpe_copies[pc].wait()
        for b in range(B):
            g_copies[pc][b].wait()
        if pc + 1 < n_pc:
            if pc >= 1:
                for b in range(B):
                    s_copies[pc - 1][b].wait()
            start_chunk(pc + 1)
        add_chunk(h)
        for b in range(B):
            row_base = b * T + pos_base + pc * C
            s_copies[pc][b] = pltpu.async_copy(
                rows_v.at[h, b], out_hbm.at[pl.ds(row_base, C)], ssems.at[h, b])
    for pc in (n_pc - 2, n_pc - 1):
        for b in range(B):
            s_copies[pc][b].wait()


def kernel(idx, wte, wpe):
    B, T = idx.shape
    V, D = wte.shape
    P = T // NW
    n_pc = P // C

    mesh = plsc.VectorSubcoreMesh(core_axis_name="c", subcore_axis_name="s")
    body = functools.partial(_emb_kernel, B, T, D, P)
    out = pl.kernel(
        body,
        out_type=jax.ShapeDtypeStruct((B * T, D), jnp.float32),
        mesh=mesh,
        scratch_types=[
            pltpu.VMEM((n_pc, B, C), jnp.int32),
            pltpu.VMEM((2, C, D), jnp.float32),
            pltpu.VMEM((2, B, C, D), jnp.float32),
            pltpu.SemaphoreType.DMA((2, B)),
            pltpu.SemaphoreType.DMA((2, B)),
            pltpu.SemaphoreType.DMA((2,)),
        ],
    )(idx.reshape(B * T), wte, wpe)
    return out.reshape(B, T, D)


# speedup vs baseline: 1.6024x; 1.6024x over previous
import functools

import jax
import jax.numpy as jnp
from jax import lax
from jax.experimental import pallas as pl
from jax.experimental.pallas import tpu as pltpu
from jax.experimental.pallas import tpu_sc as plsc

NC = 2
NS = 16
NW = NC * NS
C = 32


def _emb_kernel(B, T, D, P, idx_hbm, wpe_hbm, wte_hbm, out_hbm,
                idx_v, rows_v, wsems, gsems, ssems):
    wid = lax.axis_index("s") * NC + lax.axis_index("c")
    pos_base = wid * P
    n_chunks = P // C
    n_jobs = n_chunks * B

    for j in range(n_jobs):
        pc, b = divmod(j, B)
        row_base = b * T + pos_base + pc * C
        pltpu.sync_copy(idx_hbm.at[pl.ds(row_base, C)], idx_v.at[j])

    fills = [None] * n_jobs
    gathers = [None] * n_jobs
    stores = [None] * n_jobs

    def start_fill(j):
        pc = j // B
        fills[j] = pltpu.async_copy(
            wpe_hbm.at[pl.ds(pos_base + pc * C, C)], rows_v.at[j % 2],
            wsems.at[j % 2])

    def start_gather_add(j):
        gathers[j] = pltpu.async_copy(
            wte_hbm.at[idx_v.at[j]], rows_v.at[j % 2], gsems.at[j % 2],
            add=True)

    start_fill(0)
    fills[0].wait()
    start_gather_add(0)
    for j in range(n_jobs):
        pc, b = divmod(j, B)
        if j + 1 < n_jobs:
            if j > 0:
                stores[j - 1].wait()
            start_fill(j + 1)
        gathers[j].wait()
        row_base = b * T + pos_base + pc * C
        stores[j] = pltpu.async_copy(
            rows_v.at[j % 2], out_hbm.at[pl.ds(row_base, C)], ssems.at[j % 2])
        if j + 1 < n_jobs:
            fills[j + 1].wait()
            start_gather_add(j + 1)
    stores[n_jobs - 1].wait()


def kernel(idx, wte, wpe):
    B, T = idx.shape
    V, D = wte.shape
    P = T // NW
    n_jobs = (P // C) * B

    mesh = plsc.VectorSubcoreMesh(core_axis_name="c", subcore_axis_name="s")
    body = functools.partial(_emb_kernel, B, T, D, P)
    out = pl.kernel(
        body,
        out_type=jax.ShapeDtypeStruct((B * T, D), jnp.float32),
        mesh=mesh,
        scratch_types=[
            pltpu.VMEM((n_jobs, C), jnp.int32),
            pltpu.VMEM((2, C, D), jnp.float32),
            pltpu.SemaphoreType.DMA((2,)),
            pltpu.SemaphoreType.DMA((2,)),
            pltpu.SemaphoreType.DMA((2,)),
        ],
    )(idx.reshape(B * T), wte, wpe)
    return out.reshape(B, T, D)


# speedup vs baseline: 1.6802x; 1.0486x over previous
import functools

import jax
import jax.numpy as jnp
from jax import lax
from jax.experimental import pallas as pl
from jax.experimental.pallas import tpu as pltpu
from jax.experimental.pallas import tpu_sc as plsc

NC = 2
NS = 16
NW = NC * NS
C = 32
NBUF = 4


def _emb_kernel(B, T, D, P, idx_hbm, wpe_hbm, wte_hbm, out_hbm,
                idx_v, rows_v, wsems, gsems, ssems, isems):
    wid = lax.axis_index("s") * NC + lax.axis_index("c")
    pos_base = wid * P
    n_chunks = P // C
    n_jobs = n_chunks * B

    idx_copies = [
        pltpu.async_copy(idx_hbm.at[pl.ds(b * T + pos_base, P)], idx_v.at[b],
                         isems.at[b])
        for b in range(B)
    ]

    fills = [None] * n_jobs
    gathers = [None] * n_jobs
    stores = [None] * n_jobs

    def start_fill(j):
        pc = j // B
        fills[j] = pltpu.async_copy(
            wpe_hbm.at[pl.ds(pos_base + pc * C, C)], rows_v.at[j % NBUF],
            wsems.at[j % NBUF])

    def start_gather_add(j):
        pc, b = divmod(j, B)
        gathers[j] = pltpu.async_copy(
            wte_hbm.at[idx_v.at[b, pl.ds(pc * C, C)]], rows_v.at[j % NBUF],
            gsems.at[j % NBUF], add=True)

    for j in range(min(NBUF, n_jobs)):
        start_fill(j)
    for b in range(B):
        idx_copies[b].wait()
    for j in range(min(NBUF, n_jobs)):
        fills[j].wait()
        start_gather_add(j)
    for j in range(n_jobs):
        pc, b = divmod(j, B)
        gathers[j].wait()
        row_base = b * T + pos_base + pc * C
        stores[j] = pltpu.async_copy(
            rows_v.at[j % NBUF], out_hbm.at[pl.ds(row_base, C)],
            ssems.at[j % NBUF])
        jn = j + NBUF
        if jn < n_jobs:
            stores[j].wait()
            start_fill(jn)
            fills[jn].wait()
            start_gather_add(jn)
    for j in range(max(0, n_jobs - NBUF), n_jobs):
        stores[j].wait()


def kernel(idx, wte, wpe):
    B, T = idx.shape
    V, D = wte.shape
    P = T // NW

    mesh = plsc.VectorSubcoreMesh(core_axis_name="c", subcore_axis_name="s")
    body = functools.partial(_emb_kernel, B, T, D, P)
    out = pl.kernel(
        body,
        out_type=jax.ShapeDtypeStruct((B * T, D), jnp.float32),
        mesh=mesh,
        scratch_types=[
            pltpu.VMEM((B, P), jnp.int32),
            pltpu.VMEM((NBUF, C, D), jnp.float32),
            pltpu.SemaphoreType.DMA((NBUF,)),
            pltpu.SemaphoreType.DMA((NBUF,)),
            pltpu.SemaphoreType.DMA((NBUF,)),
            pltpu.SemaphoreType.DMA((B,)),
        ],
    )(idx.reshape(B * T), wte, wpe)
    return out.reshape(B, T, D)


# speedup vs baseline: 1.7228x; 1.0253x over previous
import functools

import jax
import jax.numpy as jnp
from jax import lax
from jax.experimental import pallas as pl
from jax.experimental.pallas import tpu as pltpu
from jax.experimental.pallas import tpu_sc as plsc

NC = 2
NS = 16
NW = NC * NS
C = 32
NBUF = 5


def _emb_kernel(B, T, D, P, idx_hbm, wpe_hbm, wte_hbm, out_hbm,
                idx_v, rows_v, wsems, gsems, ssems, isems):
    wid = lax.axis_index("s") * NC + lax.axis_index("c")
    pos_base = wid * P
    n_chunks = P // C
    n_jobs = n_chunks * B

    idx_copies = [
        pltpu.async_copy(idx_hbm.at[pl.ds(b * T + pos_base, P)], idx_v.at[b],
                         isems.at[b])
        for b in range(B)
    ]

    fills = [None] * n_jobs
    gathers = [None] * n_jobs
    stores = [None] * n_jobs

    def start_fill(j):
        pc = j // B
        fills[j] = pltpu.async_copy(
            wpe_hbm.at[pl.ds(pos_base + pc * C, C)], rows_v.at[j % NBUF],
            wsems.at[j % NBUF])

    def start_gather_add(j):
        pc, b = divmod(j, B)
        gathers[j] = pltpu.async_copy(
            wte_hbm.at[idx_v.at[b, pl.ds(pc * C, C)]], rows_v.at[j % NBUF],
            gsems.at[j % NBUF], add=True)

    for j in range(min(NBUF, n_jobs)):
        start_fill(j)
    for b in range(B):
        idx_copies[b].wait()
    for j in range(min(NBUF, n_jobs)):
        fills[j].wait()
        start_gather_add(j)
    for j in range(n_jobs):
        pc, b = divmod(j, B)
        gathers[j].wait()
        row_base = b * T + pos_base + pc * C
        stores[j] = pltpu.async_copy(
            rows_v.at[j % NBUF], out_hbm.at[pl.ds(row_base, C)],
            ssems.at[j % NBUF])
        jn = j + NBUF
        if jn < n_jobs:
            stores[j].wait()
            start_fill(jn)
            fills[jn].wait()
            start_gather_add(jn)
    for j in range(max(0, n_jobs - NBUF), n_jobs):
        stores[j].wait()


def kernel(idx, wte, wpe):
    B, T = idx.shape
    V, D = wte.shape
    P = T // NW

    mesh = plsc.VectorSubcoreMesh(core_axis_name="c", subcore_axis_name="s")
    body = functools.partial(_emb_kernel, B, T, D, P)
    out = pl.kernel(
        body,
        out_type=jax.ShapeDtypeStruct((B * T, D), jnp.float32),
        mesh=mesh,
        scratch_types=[
            pltpu.VMEM((B, P), jnp.int32),
            pltpu.VMEM((NBUF, C, D), jnp.float32),
            pltpu.SemaphoreType.DMA((NBUF,)),
            pltpu.SemaphoreType.DMA((NBUF,)),
            pltpu.SemaphoreType.DMA((NBUF,)),
            pltpu.SemaphoreType.DMA((B,)),
        ],
    )(idx.reshape(B * T), wte, wpe)
    return out.reshape(B, T, D)
